# Initial kernel scaffold; baseline (speedup 1.0000x reference)
import jax
import jax.numpy as jnp
from jax.experimental import pallas as pl


def kernel(input_ids, embed_weight, proj_weight, proj_bias):
    raise NotImplementedError("write your pallas kernel here")


# speedup vs baseline: 1.1005x; 1.1005x over previous
import functools

import jax
import jax.numpy as jnp
from jax import lax
from jax.experimental import pallas as pl
from jax.experimental.pallas import tpu as pltpu
from jax.experimental.pallas import tpu_sc as plsc

D = 64
NC = 2
NS = 16
NW = NC * NS
CHUNK = 128
K = 4
SUPER = CHUNK * K


def _gather_sc(idx3, table):
    _, n_chunks, _ = idx3.shape
    b_per_w = n_chunks * CHUNK
    n_super = b_per_w // SUPER
    N = NW * b_per_w
    mesh = plsc.VectorSubcoreMesh(core_axis_name="c", subcore_axis_name="s")

    @functools.partial(
        pl.kernel,
        mesh=mesh,
        out_type=jax.ShapeDtypeStruct((N, D), jnp.float32),
        compiler_params=pltpu.CompilerParams(use_tc_tiling_on_sc=False),
        scratch_types=[
            pltpu.VMEM((n_chunks, CHUNK), jnp.int32),
            pltpu.VMEM((SUPER, D), jnp.float32),
            pltpu.SemaphoreType.DMA,
        ],
    )
    def k(idx_hbm, table_hbm, out_hbm, idx_v, rows_v, sem):
        wid = lax.axis_index("s") * NC + lax.axis_index("c")
        base = wid * b_per_w
        pltpu.sync_copy(idx_hbm.at[wid], idx_v)

        def body(s, _):
            handles = []
            for j in range(K):
                handles.append(pltpu.async_copy(
                    table_hbm.at[idx_v.at[s * K + j]],
                    rows_v.at[pl.ds(j * CHUNK, CHUNK)],
                    sem,
                ))
            for h in handles:
                h.wait()
            pltpu.sync_copy(rows_v, out_hbm.at[pl.ds(base + s * SUPER, SUPER)])
            return 0

        lax.fori_loop(0, n_super, body, 0)

    return k(idx3, table)


def _proj_tc(x, w_t, b_row):
    N = x.shape[0]
    M = 8192
    grid = N // M

    def body(x_ref, w_ref, b_ref, o_ref):
        o_ref[...] = (
            jnp.dot(x_ref[...], w_ref[...], preferred_element_type=jnp.float32)
            + b_ref[...]
        )

    return pl.pallas_call(
        body,
        grid=(grid,),
        in_specs=[
            pl.BlockSpec((M, D), lambda i: (i, 0)),
            pl.BlockSpec((D, D), lambda i: (0, 0)),
            pl.BlockSpec((1, D), lambda i: (0, 0)),
        ],
        out_specs=pl.BlockSpec((M, D), lambda i: (i, 0)),
        out_shape=jax.ShapeDtypeStruct((N, D), jnp.float32),
    )(x, w_t, b_row)


def kernel(input_ids, embed_weight, proj_weight, proj_bias):
    B, T = input_ids.shape
    N = B * T
    idx3 = input_ids.reshape(NW, N // NW // CHUNK, CHUNK).astype(jnp.int32)
    gathered = _gather_sc(idx3, embed_weight)
    y = _proj_tc(gathered, proj_weight.T, proj_bias.reshape(1, D))
    return y.reshape(B, T, D)


# speedup vs baseline: 1.5498x; 1.4083x over previous
import functools

import jax
import jax.numpy as jnp
from jax import lax
from jax.experimental import pallas as pl
from jax.experimental.pallas import tpu as pltpu
from jax.experimental.pallas import tpu_sc as plsc

D = 64
NC = 2
NS = 16
NW = NC * NS
CHUNK = 128
K = 4
SUPER = CHUNK * K


def _gather_sc(idx3, table):
    _, n_chunks, _ = idx3.shape
    b_per_w = n_chunks * CHUNK
    n_super = b_per_w // SUPER
    N = NW * b_per_w
    mesh = plsc.VectorSubcoreMesh(core_axis_name="c", subcore_axis_name="s")

    @functools.partial(
        pl.kernel,
        mesh=mesh,
        out_type=jax.ShapeDtypeStruct((N // SUPER, SUPER, D), jnp.float32),
        compiler_params=pltpu.CompilerParams(use_tc_tiling_on_sc=False),
        scratch_types=[
            pltpu.VMEM((n_chunks, CHUNK), jnp.int32),
            pltpu.VMEM((SUPER, D), jnp.float32),
            pltpu.SemaphoreType.DMA,
        ],
    )
    def k(idx_hbm, table_hbm, out_hbm, idx_v, rows_v, sem):
        wid = lax.axis_index("s") * NC + lax.axis_index("c")
        sbase = wid * n_super
        pltpu.sync_copy(idx_hbm.at[wid], idx_v)

        def body(s, _):
            handles = []
            for j in range(K):
                handles.append(pltpu.async_copy(
                    table_hbm.at[idx_v.at[s * K + j]],
                    rows_v.at[pl.ds(j * CHUNK, CHUNK)],
                    sem,
                ))
            for h in handles:
                h.wait()
            pltpu.sync_copy(rows_v, out_hbm.at[sbase + s])
            return 0

        lax.fori_loop(0, n_super, body, 0)

    return k(idx3, table)


def _proj_tc(x3, w3, b_col, T, B):
    TB = 8
    P = B // 2

    def body(x_ref, w_ref, b_ref, o_ref):
        x = x_ref[...]
        for h in range(2):
            xh = x[:, :, h * D:(h + 1) * D]
            yh = lax.dot_general(
                w_ref[...], xh,
                dimension_numbers=(((1,), (2,)), ((0,), (0,))),
                preferred_element_type=jnp.float32,
            )
            o_ref[:, :, h * P:(h + 1) * P] = yh + b_ref[...]

    return pl.pallas_call(
        body,
        grid=(T // TB,),
        in_specs=[
            pl.BlockSpec((TB, P, 2 * D), lambda i: (i, 0, 0)),
            pl.BlockSpec((TB, D, D), lambda i: (0, 0, 0)),
            pl.BlockSpec((TB, D, 1), lambda i: (0, 0, 0)),
        ],
        out_specs=pl.BlockSpec((TB, D, B), lambda i: (i, 0, 0)),
        out_shape=jax.ShapeDtypeStruct((T, D, B), jnp.float32),
    )(x3, w3, b_col)


def kernel(input_ids, embed_weight, proj_weight, proj_bias):
    B, T = input_ids.shape
    N = B * T
    ids_perm = (
        input_ids.astype(jnp.int32)
        .reshape(2, B // 2, T)
        .transpose(2, 1, 0)
        .reshape(NW, N // NW // CHUNK, CHUNK)
    )
    gathered = _gather_sc(ids_perm, embed_weight)
    x3 = gathered.reshape(T, B // 2, 2 * D)
    w3 = jnp.broadcast_to(proj_weight.T.reshape(1, D, D), (8, D, D))
    b_col = jnp.broadcast_to(proj_bias.reshape(1, D, 1), (8, D, 1))
    y3 = _proj_tc(x3, w3, b_col, T, B)
    return y3.transpose(2, 0, 1)
